# baseline (speedup 1.0000x reference)
import functools

import jax
import jax.numpy as jnp
from jax import lax
from jax.experimental import pallas as pl
from jax.experimental.pallas import tpu as pltpu
from jax.experimental.pallas import tpu_sc as plsc

N = 10000
E = 320000
D = 128
NW = 32
EB = 128
NB = 80
EP = NW * NB * EB
NPAD = 10112
ZROWS = NPAD // 16
CROWS = NPAD // 16

@functools.cache
def _sc_kernels():
    mesh = plsc.VectorSubcoreMesh(core_axis_name="c", subcore_axis_name="s")

    DSEM = 4

    @functools.partial(
        pl.kernel,
        mesh=mesh,
        out_type=jax.ShapeDtypeStruct((2, NPAD, D), jnp.float32),
        scratch_types=[
            pltpu.VMEM((NB, EB), jnp.int32),
            pltpu.VMEM((EB, D), jnp.float32),
            pltpu.VMEM_SHARED((NPAD, D), jnp.float32),
        ]
        + [pltpu.SemaphoreType.DMA] * DSEM,
    )
    def deg_kernel(dst_hbm, ones_hbm, z_hbm, out_hbm, dst_v, ones_v, deg_sh,
                   *dsem):
        c = lax.axis_index("c")
        s = lax.axis_index("s")
        w = s * 2 + c
        pltpu.sync_copy(z_hbm.at[pl.ds(s * ZROWS, ZROWS)],
                        deg_sh.at[pl.ds(s * ZROWS, ZROWS)])
        pltpu.sync_copy(ones_hbm, ones_v)
        pltpu.sync_copy(dst_hbm.at[pl.ds(w * NB, NB)], dst_v)
        plsc.subcore_barrier()

        for b in range(DSEM):
            pltpu.async_copy(ones_v, deg_sh.at[dst_v.at[b]], dsem[b],
                             add=True)

        def body(g, carry):
            for b in range(DSEM):
                pltpu.make_async_copy(ones_v, deg_sh.at[dst_v.at[0]],
                                      dsem[b]).wait()
                pltpu.async_copy(ones_v, deg_sh.at[dst_v.at[g * DSEM + b]],
                                 dsem[b], add=True)
            return carry

        lax.fori_loop(1, NB // DSEM, body, 0)
        for b in range(DSEM):
            pltpu.make_async_copy(ones_v, deg_sh.at[dst_v.at[0]],
                                  dsem[b]).wait()
        plsc.subcore_barrier()
        pltpu.sync_copy(deg_sh.at[pl.ds(s * CROWS, CROWS)],
                        out_hbm.at[c, pl.ds(s * CROWS, CROWS)])

    RBUF = 2
    CHK = 8
    NB0 = 160
    NB1 = 0

    @functools.partial(
        pl.kernel,
        mesh=mesh,
        out_type=jax.ShapeDtypeStruct((2, NPAD, D), jnp.float32),
        scratch_types=[
            pltpu.VMEM((CHK, EB), jnp.int32),
            pltpu.VMEM((CHK, EB), jnp.int32),
            pltpu.VMEM((RBUF, EB, D), jnp.float32),
            pltpu.VMEM_SHARED((NPAD, D), jnp.float32),
        ]
        + [pltpu.SemaphoreType.DMA] * (2 * RBUF),
    )
    def agg_kernel(hs_hbm, src_hbm, dst_hbm, z_hbm, out_hbm,
                   isrc, idst, rows_v, acc_sh, *sems):
        gsem = sems[:RBUF]
        ssem = sems[RBUF:]
        c = lax.axis_index("c")
        s = lax.axis_index("s")

        def start_gather(u, b2):
            pltpu.async_copy(hs_hbm.at[isrc.at[u]], rows_v.at[b2], gsem[b2])

        def wait_gather(b2):
            pltpu.make_async_copy(hs_hbm.at[isrc.at[0]],
                                  rows_v.at[b2], gsem[b2]).wait()

        pltpu.sync_copy(z_hbm.at[pl.ds(s * ZROWS, ZROWS)],
                        acc_sh.at[pl.ds(s * ZROWS, ZROWS)])
        plsc.subcore_barrier()

        def run_blocks(base, nchk):
            def body(kc, carry):
                off = base + kc * CHK
                pltpu.sync_copy(src_hbm.at[pl.ds(off, CHK)], isrc)
                pltpu.sync_copy(dst_hbm.at[pl.ds(off, CHK)], idst)
                for b2 in range(RBUF):
                    start_gather(b2, b2)
                for u in range(CHK):
                    b2 = u % RBUF
                    wait_gather(b2)
                    pltpu.async_copy(rows_v.at[b2], acc_sh.at[idst.at[u]],
                                     ssem[b2], add=True)
                    pltpu.make_async_copy(rows_v.at[b2],
                                          acc_sh.at[idst.at[0]],
                                          ssem[b2]).wait()
                    if u + RBUF < CHK:
                        start_gather(u + RBUF, b2)
                return carry

            lax.fori_loop(0, nchk, body, 0)

        @pl.when(c == 0)
        def _():
            run_blocks(s * NB0, NB0 // CHK)

        @pl.when(c == 1)
        def _():
            run_blocks(16 * NB0 + s * NB1, NB1 // CHK)
        plsc.subcore_barrier()
        pltpu.sync_copy(acc_sh.at[pl.ds(s * CROWS, CROWS)],
                        out_hbm.at[c, pl.ds(s * CROWS, CROWS)])

    return deg_kernel, agg_kernel


_RB = 1000


def _tc_in_body(x_ref, w_ref, dp_ref, hs_ref, dis_ref):
    deg = dp_ref[0, :, 0:1] + dp_ref[1, :, 0:1] + 1.0
    dis = lax.rsqrt(deg)
    h = jnp.dot(x_ref[...], w_ref[...], preferred_element_type=jnp.float32)
    hs_ref[...] = h * dis
    dis_ref[...] = dis


def _tc_in(x, W1, degparts):
    return pl.pallas_call(
        _tc_in_body,
        grid=(N // _RB,),
        in_specs=[
            pl.BlockSpec((_RB, D), lambda i: (i, 0)),
            pl.BlockSpec((D, D), lambda i: (0, 0)),
            pl.BlockSpec((2, _RB, D), lambda i: (0, i, 0)),
        ],
        out_specs=[
            pl.BlockSpec((_RB, D), lambda i: (i, 0)),
            pl.BlockSpec((_RB, 1), lambda i: (i, 0)),
        ],
        out_shape=[
            jax.ShapeDtypeStruct((N, D), jnp.float32),
            jax.ShapeDtypeStruct((N, 1), jnp.float32),
        ],
    )(x, W1, degparts)


def _tc_mid_body(p_ref, hs_ref, dis_ref, b_ref, w_ref, out_ref):
    acc = p_ref[0] + p_ref[1] + hs_ref[...]
    z = jnp.maximum(acc * dis_ref[...] + b_ref[...], 0.0)
    out_ref[...] = jnp.dot(z, w_ref[...],
                           preferred_element_type=jnp.float32) * dis_ref[...]


def _tc_mid(parts, hs, dis, b1, W2):
    return pl.pallas_call(
        _tc_mid_body,
        grid=(N // _RB,),
        in_specs=[
            pl.BlockSpec((2, _RB, D), lambda i: (0, i, 0)),
            pl.BlockSpec((_RB, D), lambda i: (i, 0)),
            pl.BlockSpec((_RB, 1), lambda i: (i, 0)),
            pl.BlockSpec((1, D), lambda i: (0, 0)),
            pl.BlockSpec((D, D), lambda i: (0, 0)),
        ],
        out_specs=pl.BlockSpec((_RB, D), lambda i: (i, 0)),
        out_shape=jax.ShapeDtypeStruct((N, D), jnp.float32),
    )(parts, hs, dis, b1, W2)


def _tc_out_body(p_ref, hs_ref, dis_ref, b_ref, out_ref):
    acc = p_ref[0] + p_ref[1] + hs_ref[...]
    out_ref[...] = acc * dis_ref[...] + b_ref[...]


def _tc_out(parts, hs, dis, b2):
    return pl.pallas_call(
        _tc_out_body,
        grid=(N // _RB,),
        in_specs=[
            pl.BlockSpec((2, _RB, D), lambda i: (0, i, 0)),
            pl.BlockSpec((_RB, D), lambda i: (i, 0)),
            pl.BlockSpec((_RB, 1), lambda i: (i, 0)),
            pl.BlockSpec((1, D), lambda i: (0, 0)),
        ],
        out_specs=pl.BlockSpec((_RB, D), lambda i: (i, 0)),
        out_shape=jax.ShapeDtypeStruct((N, D), jnp.float32),
    )(parts, hs, dis, b2)


def kernel(x, edge_index, W1, b1, W2, b2):
    pad = EP - E
    src_p = jnp.concatenate(
        [edge_index[0], jnp.zeros((pad,), jnp.int32)]).reshape(EP // EB, EB)
    dst_p = jnp.concatenate(
        [edge_index[1], jnp.full((pad,), N, jnp.int32)]).reshape(EP // EB, EB)
    zeros_d = jnp.zeros((NPAD, D), jnp.float32)
    ones_d = jnp.ones((EB, D), jnp.float32)

    deg_kernel, agg_kernel = _sc_kernels()
    degparts = deg_kernel(dst_p, ones_d, zeros_d)
    h1s, dis = _tc_in(x, W1, degparts)
    parts1 = agg_kernel(h1s, src_p, dst_p, zeros_d)
    h2s = _tc_mid(parts1, h1s, dis, b1.reshape(1, D), W2)
    parts2 = agg_kernel(h2s, src_p, dst_p, zeros_d)
    return _tc_out(parts2, h2s, dis, b2.reshape(1, D))


# speedup vs baseline: 1.4083x; 1.4083x over previous
import functools

import jax
import jax.numpy as jnp
from jax import lax
from jax.experimental import pallas as pl
from jax.experimental.pallas import tpu as pltpu
from jax.experimental.pallas import tpu_sc as plsc

N = 10000
E = 320000
D = 128
NW = 32
EB = 128
NB = 80
EP = NW * NB * EB
NPAD = 10112
ZROWS = NPAD // 16
CROWS = NPAD // 16

@functools.cache
def _sc_kernels():
    mesh = plsc.VectorSubcoreMesh(core_axis_name="c", subcore_axis_name="s")

    DSEM = 4

    @functools.partial(
        pl.kernel,
        mesh=mesh,
        out_type=jax.ShapeDtypeStruct((2, NPAD, D), jnp.float32),
        scratch_types=[
            pltpu.VMEM((NB, EB), jnp.int32),
            pltpu.VMEM((EB, D), jnp.float32),
            pltpu.VMEM_SHARED((NPAD, D), jnp.float32),
        ]
        + [pltpu.SemaphoreType.DMA] * DSEM,
    )
    def deg_kernel(dst_hbm, ones_hbm, z_hbm, out_hbm, dst_v, ones_v, deg_sh,
                   *dsem):
        c = lax.axis_index("c")
        s = lax.axis_index("s")
        w = s * 2 + c
        pltpu.sync_copy(z_hbm.at[pl.ds(s * ZROWS, ZROWS)],
                        deg_sh.at[pl.ds(s * ZROWS, ZROWS)])
        pltpu.sync_copy(ones_hbm, ones_v)
        pltpu.sync_copy(dst_hbm.at[pl.ds(w * NB, NB)], dst_v)
        plsc.subcore_barrier()

        for b in range(DSEM):
            pltpu.async_copy(ones_v, deg_sh.at[dst_v.at[b]], dsem[b],
                             add=True)

        def body(g, carry):
            for b in range(DSEM):
                pltpu.make_async_copy(ones_v, deg_sh.at[dst_v.at[0]],
                                      dsem[b]).wait()
                pltpu.async_copy(ones_v, deg_sh.at[dst_v.at[g * DSEM + b]],
                                 dsem[b], add=True)
            return carry

        lax.fori_loop(1, NB // DSEM, body, 0)
        for b in range(DSEM):
            pltpu.make_async_copy(ones_v, deg_sh.at[dst_v.at[0]],
                                  dsem[b]).wait()
        plsc.subcore_barrier()
        pltpu.sync_copy(deg_sh.at[pl.ds(s * CROWS, CROWS)],
                        out_hbm.at[c, pl.ds(s * CROWS, CROWS)])

    RBUF = 2
    CHK = 8
    NB0 = 144
    NB1 = 16

    @functools.partial(
        pl.kernel,
        mesh=mesh,
        out_type=jax.ShapeDtypeStruct((2, NPAD, D), jnp.float32),
        scratch_types=[
            pltpu.VMEM((CHK, EB), jnp.int32),
            pltpu.VMEM((CHK, EB), jnp.int32),
            pltpu.VMEM((RBUF, EB, D), jnp.float32),
            pltpu.VMEM_SHARED((NPAD, D), jnp.float32),
        ]
        + [pltpu.SemaphoreType.DMA] * (2 * RBUF),
    )
    def agg_kernel(hs_hbm, src_hbm, dst_hbm, z_hbm, out_hbm,
                   isrc, idst, rows_v, acc_sh, *sems):
        gsem = sems[:RBUF]
        ssem = sems[RBUF:]
        c = lax.axis_index("c")
        s = lax.axis_index("s")

        def start_gather(u, b2):
            pltpu.async_copy(hs_hbm.at[isrc.at[u]], rows_v.at[b2], gsem[b2])

        def wait_gather(b2):
            pltpu.make_async_copy(hs_hbm.at[isrc.at[0]],
                                  rows_v.at[b2], gsem[b2]).wait()

        pltpu.sync_copy(z_hbm.at[pl.ds(s * ZROWS, ZROWS)],
                        acc_sh.at[pl.ds(s * ZROWS, ZROWS)])
        plsc.subcore_barrier()

        def run_blocks(base, nchk):
            def body(kc, carry):
                off = base + kc * CHK
                pltpu.sync_copy(src_hbm.at[pl.ds(off, CHK)], isrc)
                pltpu.sync_copy(dst_hbm.at[pl.ds(off, CHK)], idst)
                for b2 in range(RBUF):
                    start_gather(b2, b2)
                for u in range(CHK):
                    b2 = u % RBUF
                    wait_gather(b2)
                    pltpu.async_copy(rows_v.at[b2], acc_sh.at[idst.at[u]],
                                     ssem[b2], add=True)
                    pltpu.make_async_copy(rows_v.at[b2],
                                          acc_sh.at[idst.at[0]],
                                          ssem[b2]).wait()
                    if u + RBUF < CHK:
                        start_gather(u + RBUF, b2)
                return carry

            lax.fori_loop(0, nchk, body, 0)

        @pl.when(c == 0)
        def _():
            run_blocks(s * NB0, NB0 // CHK)

        @pl.when(c == 1)
        def _():
            run_blocks(16 * NB0 + s * NB1, NB1 // CHK)
        plsc.subcore_barrier()
        pltpu.sync_copy(acc_sh.at[pl.ds(s * CROWS, CROWS)],
                        out_hbm.at[c, pl.ds(s * CROWS, CROWS)])

    return deg_kernel, agg_kernel


_RB = 1000


def _tc_in_body(x_ref, w_ref, dp_ref, hs_ref, dis_ref):
    deg = dp_ref[0, :, 0:1] + dp_ref[1, :, 0:1] + 1.0
    dis = lax.rsqrt(deg)
    h = jnp.dot(x_ref[...], w_ref[...], preferred_element_type=jnp.float32)
    hs_ref[...] = h * dis
    dis_ref[...] = dis


def _tc_in(x, W1, degparts):
    return pl.pallas_call(
        _tc_in_body,
        grid=(N // _RB,),
        in_specs=[
            pl.BlockSpec((_RB, D), lambda i: (i, 0)),
            pl.BlockSpec((D, D), lambda i: (0, 0)),
            pl.BlockSpec((2, _RB, D), lambda i: (0, i, 0)),
        ],
        out_specs=[
            pl.BlockSpec((_RB, D), lambda i: (i, 0)),
            pl.BlockSpec((_RB, 1), lambda i: (i, 0)),
        ],
        out_shape=[
            jax.ShapeDtypeStruct((N, D), jnp.float32),
            jax.ShapeDtypeStruct((N, 1), jnp.float32),
        ],
    )(x, W1, degparts)


def _tc_mid_body(p_ref, hs_ref, dis_ref, b_ref, w_ref, out_ref):
    acc = p_ref[0] + p_ref[1] + hs_ref[...]
    z = jnp.maximum(acc * dis_ref[...] + b_ref[...], 0.0)
    out_ref[...] = jnp.dot(z, w_ref[...],
                           preferred_element_type=jnp.float32) * dis_ref[...]


def _tc_mid(parts, hs, dis, b1, W2):
    return pl.pallas_call(
        _tc_mid_body,
        grid=(N // _RB,),
        in_specs=[
            pl.BlockSpec((2, _RB, D), lambda i: (0, i, 0)),
            pl.BlockSpec((_RB, D), lambda i: (i, 0)),
            pl.BlockSpec((_RB, 1), lambda i: (i, 0)),
            pl.BlockSpec((1, D), lambda i: (0, 0)),
            pl.BlockSpec((D, D), lambda i: (0, 0)),
        ],
        out_specs=pl.BlockSpec((_RB, D), lambda i: (i, 0)),
        out_shape=jax.ShapeDtypeStruct((N, D), jnp.float32),
    )(parts, hs, dis, b1, W2)


def _tc_out_body(p_ref, hs_ref, dis_ref, b_ref, out_ref):
    acc = p_ref[0] + p_ref[1] + hs_ref[...]
    out_ref[...] = acc * dis_ref[...] + b_ref[...]


def _tc_out(parts, hs, dis, b2):
    return pl.pallas_call(
        _tc_out_body,
        grid=(N // _RB,),
        in_specs=[
            pl.BlockSpec((2, _RB, D), lambda i: (0, i, 0)),
            pl.BlockSpec((_RB, D), lambda i: (i, 0)),
            pl.BlockSpec((_RB, 1), lambda i: (i, 0)),
            pl.BlockSpec((1, D), lambda i: (0, 0)),
        ],
        out_specs=pl.BlockSpec((_RB, D), lambda i: (i, 0)),
        out_shape=jax.ShapeDtypeStruct((N, D), jnp.float32),
    )(parts, hs, dis, b2)


def kernel(x, edge_index, W1, b1, W2, b2):
    pad = EP - E
    src_p = jnp.concatenate(
        [edge_index[0], jnp.zeros((pad,), jnp.int32)]).reshape(EP // EB, EB)
    dst_p = jnp.concatenate(
        [edge_index[1], jnp.full((pad,), N, jnp.int32)]).reshape(EP // EB, EB)
    zeros_d = jnp.zeros((NPAD, D), jnp.float32)
    ones_d = jnp.ones((EB, D), jnp.float32)

    deg_kernel, agg_kernel = _sc_kernels()
    degparts = deg_kernel(dst_p, ones_d, zeros_d)
    h1s, dis = _tc_in(x, W1, degparts)
    parts1 = agg_kernel(h1s, src_p, dst_p, zeros_d)
    h2s = _tc_mid(parts1, h1s, dis, b1.reshape(1, D), W2)
    parts2 = agg_kernel(h2s, src_p, dst_p, zeros_d)
    return _tc_out(parts2, h2s, dis, b2.reshape(1, D))


# speedup vs baseline: 1.4282x; 1.0141x over previous
import functools

import jax
import jax.numpy as jnp
from jax import lax
from jax.experimental import pallas as pl
from jax.experimental.pallas import tpu as pltpu
from jax.experimental.pallas import tpu_sc as plsc

N = 10000
E = 320000
D = 128
NW = 32
EB = 128
NB = 80
EP = NW * NB * EB
NPAD = 10112
ZROWS = NPAD // 16
CROWS = NPAD // 16

@functools.cache
def _sc_kernels():
    mesh = plsc.VectorSubcoreMesh(core_axis_name="c", subcore_axis_name="s")

    DSEM = 4

    @functools.partial(
        pl.kernel,
        mesh=mesh,
        out_type=jax.ShapeDtypeStruct((2, NPAD, D), jnp.float32),
        scratch_types=[
            pltpu.VMEM((NB, EB), jnp.int32),
            pltpu.VMEM((EB, D), jnp.float32),
            pltpu.VMEM_SHARED((NPAD, D), jnp.float32),
        ]
        + [pltpu.SemaphoreType.DMA] * DSEM,
    )
    def deg_kernel(dst_hbm, ones_hbm, z_hbm, out_hbm, dst_v, ones_v, deg_sh,
                   *dsem):
        c = lax.axis_index("c")
        s = lax.axis_index("s")
        w = s * 2 + c
        pltpu.sync_copy(z_hbm.at[pl.ds(s * ZROWS, ZROWS)],
                        deg_sh.at[pl.ds(s * ZROWS, ZROWS)])
        pltpu.sync_copy(ones_hbm, ones_v)
        pltpu.sync_copy(dst_hbm.at[pl.ds(w * NB, NB)], dst_v)
        plsc.subcore_barrier()

        for b in range(DSEM):
            pltpu.async_copy(ones_v, deg_sh.at[dst_v.at[b]], dsem[b],
                             add=True)

        def body(g, carry):
            for b in range(DSEM):
                pltpu.make_async_copy(ones_v, deg_sh.at[dst_v.at[0]],
                                      dsem[b]).wait()
                pltpu.async_copy(ones_v, deg_sh.at[dst_v.at[g * DSEM + b]],
                                 dsem[b], add=True)
            return carry

        lax.fori_loop(1, NB // DSEM, body, 0)
        for b in range(DSEM):
            pltpu.make_async_copy(ones_v, deg_sh.at[dst_v.at[0]],
                                  dsem[b]).wait()
        plsc.subcore_barrier()
        pltpu.sync_copy(deg_sh.at[pl.ds(s * CROWS, CROWS)],
                        out_hbm.at[c, pl.ds(s * CROWS, CROWS)])

    RBUF = 2
    CHK = 8
    NB0 = 152
    NB1 = 8

    @functools.partial(
        pl.kernel,
        mesh=mesh,
        out_type=jax.ShapeDtypeStruct((2, NPAD, D), jnp.float32),
        scratch_types=[
            pltpu.VMEM((CHK, EB), jnp.int32),
            pltpu.VMEM((CHK, EB), jnp.int32),
            pltpu.VMEM((RBUF, EB, D), jnp.float32),
            pltpu.VMEM_SHARED((NPAD, D), jnp.float32),
        ]
        + [pltpu.SemaphoreType.DMA] * (2 * RBUF),
    )
    def agg_kernel(hs_hbm, src_hbm, dst_hbm, z_hbm, out_hbm,
                   isrc, idst, rows_v, acc_sh, *sems):
        gsem = sems[:RBUF]
        ssem = sems[RBUF:]
        c = lax.axis_index("c")
        s = lax.axis_index("s")

        def start_gather(u, b2):
            pltpu.async_copy(hs_hbm.at[isrc.at[u]], rows_v.at[b2], gsem[b2])

        def wait_gather(b2):
            pltpu.make_async_copy(hs_hbm.at[isrc.at[0]],
                                  rows_v.at[b2], gsem[b2]).wait()

        pltpu.sync_copy(z_hbm.at[pl.ds(s * ZROWS, ZROWS)],
                        acc_sh.at[pl.ds(s * ZROWS, ZROWS)])
        plsc.subcore_barrier()

        def run_blocks(base, nchk):
            def body(kc, carry):
                off = base + kc * CHK
                pltpu.sync_copy(src_hbm.at[pl.ds(off, CHK)], isrc)
                pltpu.sync_copy(dst_hbm.at[pl.ds(off, CHK)], idst)
                for b2 in range(RBUF):
                    start_gather(b2, b2)
                for u in range(CHK):
                    b2 = u % RBUF
                    wait_gather(b2)
                    pltpu.async_copy(rows_v.at[b2], acc_sh.at[idst.at[u]],
                                     ssem[b2], add=True)
                    pltpu.make_async_copy(rows_v.at[b2],
                                          acc_sh.at[idst.at[0]],
                                          ssem[b2]).wait()
                    if u + RBUF < CHK:
                        start_gather(u + RBUF, b2)
                return carry

            lax.fori_loop(0, nchk, body, 0)

        @pl.when(c == 0)
        def _():
            run_blocks(s * NB0, NB0 // CHK)

        @pl.when(c == 1)
        def _():
            run_blocks(16 * NB0 + s * NB1, NB1 // CHK)
        plsc.subcore_barrier()
        pltpu.sync_copy(acc_sh.at[pl.ds(s * CROWS, CROWS)],
                        out_hbm.at[c, pl.ds(s * CROWS, CROWS)])

    return deg_kernel, agg_kernel


_RB = 1000


def _tc_in_body(x_ref, w_ref, dp_ref, hs_ref, dis_ref):
    deg = dp_ref[0, :, 0:1] + dp_ref[1, :, 0:1] + 1.0
    dis = lax.rsqrt(deg)
    h = jnp.dot(x_ref[...], w_ref[...], preferred_element_type=jnp.float32)
    hs_ref[...] = h * dis
    dis_ref[...] = dis


def _tc_in(x, W1, degparts):
    return pl.pallas_call(
        _tc_in_body,
        grid=(N // _RB,),
        in_specs=[
            pl.BlockSpec((_RB, D), lambda i: (i, 0)),
            pl.BlockSpec((D, D), lambda i: (0, 0)),
            pl.BlockSpec((2, _RB, D), lambda i: (0, i, 0)),
        ],
        out_specs=[
            pl.BlockSpec((_RB, D), lambda i: (i, 0)),
            pl.BlockSpec((_RB, 1), lambda i: (i, 0)),
        ],
        out_shape=[
            jax.ShapeDtypeStruct((N, D), jnp.float32),
            jax.ShapeDtypeStruct((N, 1), jnp.float32),
        ],
    )(x, W1, degparts)


def _tc_mid_body(p_ref, hs_ref, dis_ref, b_ref, w_ref, out_ref):
    acc = p_ref[0] + p_ref[1] + hs_ref[...]
    z = jnp.maximum(acc * dis_ref[...] + b_ref[...], 0.0)
    out_ref[...] = jnp.dot(z, w_ref[...],
                           preferred_element_type=jnp.float32) * dis_ref[...]


def _tc_mid(parts, hs, dis, b1, W2):
    return pl.pallas_call(
        _tc_mid_body,
        grid=(N // _RB,),
        in_specs=[
            pl.BlockSpec((2, _RB, D), lambda i: (0, i, 0)),
            pl.BlockSpec((_RB, D), lambda i: (i, 0)),
            pl.BlockSpec((_RB, 1), lambda i: (i, 0)),
            pl.BlockSpec((1, D), lambda i: (0, 0)),
            pl.BlockSpec((D, D), lambda i: (0, 0)),
        ],
        out_specs=pl.BlockSpec((_RB, D), lambda i: (i, 0)),
        out_shape=jax.ShapeDtypeStruct((N, D), jnp.float32),
    )(parts, hs, dis, b1, W2)


def _tc_out_body(p_ref, hs_ref, dis_ref, b_ref, out_ref):
    acc = p_ref[0] + p_ref[1] + hs_ref[...]
    out_ref[...] = acc * dis_ref[...] + b_ref[...]


def _tc_out(parts, hs, dis, b2):
    return pl.pallas_call(
        _tc_out_body,
        grid=(N // _RB,),
        in_specs=[
            pl.BlockSpec((2, _RB, D), lambda i: (0, i, 0)),
            pl.BlockSpec((_RB, D), lambda i: (i, 0)),
            pl.BlockSpec((_RB, 1), lambda i: (i, 0)),
            pl.BlockSpec((1, D), lambda i: (0, 0)),
        ],
        out_specs=pl.BlockSpec((_RB, D), lambda i: (i, 0)),
        out_shape=jax.ShapeDtypeStruct((N, D), jnp.float32),
    )(parts, hs, dis, b2)


def kernel(x, edge_index, W1, b1, W2, b2):
    pad = EP - E
    src_p = jnp.concatenate(
        [edge_index[0], jnp.zeros((pad,), jnp.int32)]).reshape(EP // EB, EB)
    dst_p = jnp.concatenate(
        [edge_index[1], jnp.full((pad,), N, jnp.int32)]).reshape(EP // EB, EB)
    zeros_d = jnp.zeros((NPAD, D), jnp.float32)
    ones_d = jnp.ones((EB, D), jnp.float32)

    deg_kernel, agg_kernel = _sc_kernels()
    degparts = deg_kernel(dst_p, ones_d, zeros_d)
    h1s, dis = _tc_in(x, W1, degparts)
    parts1 = agg_kernel(h1s, src_p, dst_p, zeros_d)
    h2s = _tc_mid(parts1, h1s, dis, b1.reshape(1, D), W2)
    parts2 = agg_kernel(h2s, src_p, dst_p, zeros_d)
    return _tc_out(parts2, h2s, dis, b2.reshape(1, D))
